# baseline (speedup 1.0000x reference)
import functools

import jax
import jax.numpy as jnp
from jax import lax
from jax.experimental import pallas as pl
from jax.experimental.pallas import tpu as pltpu
from jax.experimental.pallas import tpu_sc as plsc

_NUM_CORES = 2
_NUM_WORKERS = 32
_BATCH = 16384
_DIM = 32

_RANGE = 31232
_MAIN_END = _RANGE * _NUM_WORKERS
_WAVE = 1024
_NFULL = 30
_LANES = 16
_NGRP = _BATCH // _LANES
_EB = 4
_OUT_ROWS = _BATCH + _NUM_WORKERS * _LANES


@functools.partial(
    pl.kernel,
    mesh=plsc.VectorSubcoreMesh(core_axis_name="c", subcore_axis_name="s"),
    out_type=jax.ShapeDtypeStruct((_OUT_ROWS, 128), jnp.float32),
    scratch_types=[
        pltpu.VMEM((_BATCH,), jnp.int32),
        pltpu.VMEM((_BATCH,), jnp.int32),
        pltpu.VMEM((_BATCH,), jnp.int32),
        pltpu.VMEM((2, 8, _DIM, 128), jnp.float32),
        pltpu.VMEM((_EB, _LANES, 128), jnp.float32),
        pltpu.VMEM((_EB * _LANES,), jnp.int32),
        pltpu.SemaphoreType.DMA,
        pltpu.SemaphoreType.DMA,
    ],
    compiler_params=pltpu.CompilerParams(
        use_tc_tiling_on_sc=True, needs_layout_passes=False
    ),
)
def _gather_kernel(
    idx_hbm, table_t_hbm, out_hbm,
    idx_v, own_r, own_j, wave_v, ring_v, jring_v, sem_in, sem_out,
):
    wid = lax.axis_index("s") * _NUM_CORES + lax.axis_index("c")
    lane = lax.iota(jnp.int32, _LANES)
    r0 = wid * _RANGE
    xbase = jnp.where(wid > 4, 0, _MAIN_END + wid * 128)

    def wave_src(base, tc):
        return table_t_hbm.at[
            :, pl.ds(pl.multiple_of(base, 128) + tc * 128, 128)
        ]

    def fire_wave(base, b, ncols):
        for tc in range(ncols):
            pltpu.async_copy(wave_src(base, tc), wave_v.at[b, tc], sem_in)

    def wait_wave(base, b, ncols):
        for tc in range(ncols):
            pltpu.make_async_copy(
                wave_src(base, tc), wave_v.at[b, tc], sem_in
            ).wait()

    fire_wave(r0, 0, 8)
    fire_wave(r0 + _WAVE, 1, 8)

    pltpu.sync_copy(idx_hbm, idx_v)

    def sel_body(g, cnt):
        v = idx_v[pl.ds(g * _LANES, _LANES)]
        d = v - _MAIN_END
        own = ((v >= r0) & (v < r0 + _RANGE)) | ((d >= 0) & ((d >> 7) == wid))
        plsc.store_compressed(own_r.at[pl.ds(cnt, _LANES)], v, mask=own)
        plsc.store_compressed(
            own_j.at[pl.ds(cnt, _LANES)], g * _LANES + lane, mask=own
        )
        pc = plsc.all_reduce_population_count(own)
        return cnt + lax.squeeze(lax.slice(pc, (0,), (1,)), (0,))

    cnt = lax.fori_loop(0, _NGRP, sel_body, jnp.int32(0))
    ngroups = (cnt + _LANES - 1) >> 4

    def ring_slab(cb):
        slot = cb & (_EB - 1)
        return (
            ring_v.at[slot],
            out_hbm.at[jring_v.at[pl.ds(slot * _LANES, _LANES)]],
        )

    def process_wave(b, wb, wn, carry):
        def grp_body(e, c):
            egc, waits = c
            gidx = e * _LANES + lane
            rv = own_r[pl.ds(e * _LANES, _LANES)]
            jv = own_j[pl.ds(e * _LANES, _LANES)]
            inw = (rv >= wb) & (rv < wb + wn) & (gidx < cnt)
            npc_v = plsc.all_reduce_population_count(inw)
            npc = lax.squeeze(lax.slice(npc_v, (0,), (1,)), (0,))
            fill = egc & (_LANES - 1)
            cb_old = egc >> 4
            need_wait = (fill + npc > _LANES) & (cb_old + 1 >= _EB)

            @pl.when(need_wait)
            def _wait_ring():
                src, dst = ring_slab(cb_old + 1)
                pltpu.make_async_copy(src, dst, sem_out).wait()

            @pl.when(npc > 0)
            def _extract():
                rank = plsc.cumsum(inw.astype(jnp.int32)) - 1
                pos = egc + rank
                slotv = (pos >> 4) & (_EB - 1)
                rowv = pos & (_LANES - 1)
                local = (rv - wb) & (_WAVE - 1)
                tcv = (local >> 7) & 7
                lv = local & 127
                bvec = lane * 0 + b
                for c in range(_DIM):
                    cvec = (lane + c) & (_DIM - 1)
                    vals = plsc.load_gather(
                        wave_v, [bvec, tcv, cvec, lv], mask=inw
                    )
                    plsc.store_scatter(
                        ring_v, [slotv, rowv, cvec], vals, mask=inw
                    )
                plsc.store_scatter(
                    jring_v, [pos & (_EB * _LANES - 1)], jv, mask=inw
                )

            @pl.when(fill + npc >= _LANES)
            def _fire_block():
                src, dst = ring_slab(cb_old)
                pltpu.async_copy(src, dst, sem_out)

            new_waits = jnp.where(need_wait, waits + 1, waits)
            return (egc + npc, new_waits)

        return lax.fori_loop(0, ngroups, grp_body, carry)

    def wave_body(k, carry):
        b = k & 1
        wb = r0 + k * _WAVE
        wait_wave(wb, b, 8)
        carry = process_wave(b, wb, _WAVE, carry)

        @pl.when(k < _NFULL - 2)
        def _prefetch():
            fire_wave(r0 + (k + 2) * _WAVE, b, 8)

        return carry

    carry = lax.fori_loop(0, _NFULL, wave_body, (jnp.int32(0), jnp.int32(0)))

    wb30 = r0 + _NFULL * _WAVE
    fire_wave(wb30, 0, 4)
    wait_wave(wb30, 0, 4)
    carry = process_wave(0, wb30, 4 * 128, carry)

    fire_wave(xbase, 1, 1)
    wait_wave(xbase, 1, 1)
    carry = process_wave(1, xbase, 128, carry)
    egc, waits = carry

    left = egc & (_LANES - 1)

    @pl.when(left > 0)
    def _flush():
        cb = egc >> 4
        slot = cb & (_EB - 1)
        jv_tail = jring_v[pl.ds(slot * _LANES, _LANES)]
        dummy = _BATCH + wid * _LANES + lane
        jring_v[pl.ds(slot * _LANES, _LANES)] = jnp.where(
            lane < left, jv_tail, dummy
        )
        src, dst = ring_slab(cb)
        pltpu.async_copy(src, dst, sem_out)

    total_fired = (egc >> 4) + jnp.where(left > 0, 1, 0).astype(jnp.int32)
    outstanding = total_fired - waits

    def drain_body(d, acc):
        src, dst = ring_slab(jnp.int32(0))
        pltpu.make_async_copy(src, dst, sem_out).wait()
        return acc

    lax.fori_loop(0, outstanding, drain_body, jnp.int32(0))


def kernel(shape_idx, emb_table):
    out_p = _gather_kernel(shape_idx.astype(jnp.int32), emb_table.T)
    return out_p[:_BATCH, :_DIM]


# speedup vs baseline: 1.1530x; 1.1530x over previous
import functools

import jax
import jax.numpy as jnp
from jax import lax
from jax.experimental import pallas as pl
from jax.experimental.pallas import tpu as pltpu
from jax.experimental.pallas import tpu_sc as plsc

_NUM_CORES = 2
_NUM_WORKERS = 32
_BATCH = 16384
_DIM = 32

_RANGE = 31232
_MAIN_END = _RANGE * _NUM_WORKERS
_WAVE = 1024
_NFULL = 30
_LANES = 16
_NGRP = _BATCH // _LANES
_EB = 4
_OUT_ROWS = _BATCH + _NUM_WORKERS * _LANES


@functools.partial(
    pl.kernel,
    mesh=plsc.VectorSubcoreMesh(core_axis_name="c", subcore_axis_name="s"),
    out_type=jax.ShapeDtypeStruct((_OUT_ROWS, 128), jnp.float32),
    scratch_types=[
        pltpu.VMEM((_BATCH,), jnp.int32),
        pltpu.VMEM((_BATCH,), jnp.int32),
        pltpu.VMEM((_BATCH,), jnp.int32),
        pltpu.VMEM((2, 8, _DIM, 128), jnp.float32),
        pltpu.VMEM((_EB, _LANES, 128), jnp.float32),
        pltpu.VMEM((_EB * _LANES,), jnp.int32),
        pltpu.SemaphoreType.DMA,
        pltpu.SemaphoreType.DMA,
    ],
    compiler_params=pltpu.CompilerParams(
        use_tc_tiling_on_sc=True, needs_layout_passes=False
    ),
)
def _gather_kernel(
    idx_hbm, table_t_hbm, out_hbm,
    idx_v, own_r, own_j, wave_v, ring_v, jring_v, sem_in, sem_out,
):
    wid = lax.axis_index("s") * _NUM_CORES + lax.axis_index("c")
    lane = lax.iota(jnp.int32, _LANES)
    r0 = wid * _RANGE
    xbase = jnp.where(wid > 4, 0, _MAIN_END + wid * 128)

    def wave_src(base, tc):
        return table_t_hbm.at[
            :, pl.ds(pl.multiple_of(base, 128) + tc * 128, 128)
        ]

    def fire_wave(base, b, ncols):
        for tc in range(ncols):
            pltpu.async_copy(wave_src(base, tc), wave_v.at[b, tc], sem_in)

    def wait_wave(base, b, ncols):
        for tc in range(ncols):
            pltpu.make_async_copy(
                wave_src(base, tc), wave_v.at[b, tc], sem_in
            ).wait()

    fire_wave(r0, 0, 8)
    fire_wave(r0 + _WAVE, 1, 8)

    pltpu.sync_copy(idx_hbm, idx_v)

    def sel_body(g, cnt):
        v = idx_v[pl.ds(g * _LANES, _LANES)]
        d = v - _MAIN_END
        own = ((v >= r0) & (v < r0 + _RANGE)) | ((d >= 0) & ((d >> 7) == wid))
        plsc.store_compressed(own_r.at[pl.ds(cnt, _LANES)], v, mask=own)
        plsc.store_compressed(
            own_j.at[pl.ds(cnt, _LANES)], g * _LANES + lane, mask=own
        )
        pc = plsc.all_reduce_population_count(own)
        return cnt + lax.squeeze(lax.slice(pc, (0,), (1,)), (0,))

    cnt = lax.fori_loop(0, _NGRP, sel_body, jnp.int32(0))
    ngroups = (cnt + _LANES - 1) >> 4

    def ring_slab(cb):
        slot = cb & (_EB - 1)
        return (
            ring_v.at[slot],
            out_hbm.at[jring_v.at[pl.ds(slot * _LANES, _LANES)]],
        )

    def process_wave(b, wb, wn, carry):
        def grp_body(e, c):
            egc, waits = c
            gidx = e * _LANES + lane
            rv = own_r[pl.ds(e * _LANES, _LANES)]
            jv = own_j[pl.ds(e * _LANES, _LANES)]
            inw = (rv >= wb) & (rv < wb + wn) & (gidx < cnt)
            npc_v = plsc.all_reduce_population_count(inw)
            npc = lax.squeeze(lax.slice(npc_v, (0,), (1,)), (0,))
            fill = egc & (_LANES - 1)
            cb_old = egc >> 4
            need_wait = (fill + npc > _LANES) & (cb_old + 1 >= _EB)

            @pl.when(need_wait)
            def _wait_ring():
                src, dst = ring_slab(cb_old + 1)
                pltpu.make_async_copy(src, dst, sem_out).wait()

            @pl.when(npc > 0)
            def _extract():
                rank = plsc.cumsum(inw.astype(jnp.int32)) - 1
                pos = egc + rank
                slotv = (pos >> 4) & (_EB - 1)
                rowv = pos & (_LANES - 1)
                local = (rv - wb) & (_WAVE - 1)
                tcv = (local >> 7) & 7
                lv = local & 127
                bvec = lane * 0 + b
                for c in range(_DIM):
                    cvec = lane * 0 + c
                    vals = plsc.load_gather(
                        wave_v, [bvec, tcv, cvec, lv], mask=inw
                    )
                    plsc.store_scatter(
                        ring_v, [slotv, rowv, cvec], vals, mask=inw
                    )
                plsc.store_scatter(
                    jring_v, [pos & (_EB * _LANES - 1)], jv, mask=inw
                )

            @pl.when(fill + npc >= _LANES)
            def _fire_block():
                src, dst = ring_slab(cb_old)
                pltpu.async_copy(src, dst, sem_out)

            new_waits = jnp.where(need_wait, waits + 1, waits)
            return (egc + npc, new_waits)

        return lax.fori_loop(0, ngroups, grp_body, carry)

    def wave_body(k, carry):
        b = k & 1
        wb = r0 + k * _WAVE
        wait_wave(wb, b, 8)
        carry = process_wave(b, wb, _WAVE, carry)

        @pl.when(k < _NFULL - 2)
        def _prefetch():
            fire_wave(r0 + (k + 2) * _WAVE, b, 8)

        return carry

    carry = lax.fori_loop(0, _NFULL, wave_body, (jnp.int32(0), jnp.int32(0)))

    wb30 = r0 + _NFULL * _WAVE
    fire_wave(wb30, 0, 4)
    wait_wave(wb30, 0, 4)
    carry = process_wave(0, wb30, 4 * 128, carry)

    fire_wave(xbase, 1, 1)
    wait_wave(xbase, 1, 1)
    carry = process_wave(1, xbase, 128, carry)
    egc, waits = carry

    left = egc & (_LANES - 1)

    @pl.when(left > 0)
    def _flush():
        cb = egc >> 4
        slot = cb & (_EB - 1)
        jv_tail = jring_v[pl.ds(slot * _LANES, _LANES)]
        dummy = _BATCH + wid * _LANES + lane
        jring_v[pl.ds(slot * _LANES, _LANES)] = jnp.where(
            lane < left, jv_tail, dummy
        )
        src, dst = ring_slab(cb)
        pltpu.async_copy(src, dst, sem_out)

    total_fired = (egc >> 4) + jnp.where(left > 0, 1, 0).astype(jnp.int32)
    outstanding = total_fired - waits

    def drain_body(d, acc):
        src, dst = ring_slab(jnp.int32(0))
        pltpu.make_async_copy(src, dst, sem_out).wait()
        return acc

    lax.fori_loop(0, outstanding, drain_body, jnp.int32(0))


def kernel(shape_idx, emb_table):
    out_p = _gather_kernel(shape_idx.astype(jnp.int32), emb_table.T)
    return out_p[:_BATCH, :_DIM]


# speedup vs baseline: 1.2862x; 1.1156x over previous
import functools

import jax
import jax.numpy as jnp
from jax import lax
from jax.experimental import pallas as pl
from jax.experimental.pallas import tpu as pltpu
from jax.experimental.pallas import tpu_sc as plsc

_NUM_CORES = 2
_NUM_WORKERS = 32
_BATCH = 16384
_DIM = 32

_RANGE = 31232
_MAIN_END = _RANGE * _NUM_WORKERS
_XREL = _RANGE
_WCOLS = 12
_WAVE = _WCOLS * 128
_NFULL = 20
_LANES = 16
_ICHUNK = 4096
_EB = 4
_OUT_ROWS = _BATCH + _NUM_WORKERS * _LANES
_SENTINEL = jnp.int32(0x7FFFFFFF)


@functools.partial(
    pl.kernel,
    mesh=plsc.VectorSubcoreMesh(core_axis_name="c", subcore_axis_name="s"),
    out_type=jax.ShapeDtypeStruct((_OUT_ROWS, 128), jnp.float32),
    scratch_types=[
        pltpu.VMEM((_ICHUNK,), jnp.int32),
        pltpu.VMEM((_BATCH + _LANES,), jnp.int32),
        pltpu.VMEM((2, _WCOLS, _DIM, 128), jnp.float32),
        pltpu.VMEM((_EB, _LANES, 128), jnp.float32),
        pltpu.VMEM((_EB * _LANES,), jnp.int32),
        pltpu.SemaphoreType.DMA,
        pltpu.SemaphoreType.DMA,
    ],
    compiler_params=pltpu.CompilerParams(
        use_tc_tiling_on_sc=True, needs_layout_passes=False
    ),
)
def _gather_kernel(
    idx_hbm, table_t_hbm, out_hbm,
    ichunk_v, own_v, wave_v, ring_v, jring_v, sem_in, sem_out,
):
    wid = lax.axis_index("s") * _NUM_CORES + lax.axis_index("c")
    lane = lax.iota(jnp.int32, _LANES)
    r0 = wid * _RANGE
    xbase = jnp.where(wid > 4, 0, _MAIN_END + wid * 128)

    def wave_src(base, tc):
        return table_t_hbm.at[
            :, pl.ds(pl.multiple_of(base, 128) + tc * 128, 128)
        ]

    def fire_wave(base, b, ncols):
        for tc in range(ncols):
            pltpu.async_copy(wave_src(base, tc), wave_v.at[b, tc], sem_in)

    def wait_wave(base, b, ncols):
        for tc in range(ncols):
            pltpu.make_async_copy(
                wave_src(base, tc), wave_v.at[b, tc], sem_in
            ).wait()

    fire_wave(r0, 0, _WCOLS)
    fire_wave(r0 + _WAVE, 1, _WCOLS)

    def sel_chunk(c, cnt):
        pltpu.sync_copy(idx_hbm.at[pl.ds(c * _ICHUNK, _ICHUNK)], ichunk_v)

        def sel_body(g, cnt):
            v = ichunk_v[pl.ds(g * _LANES, _LANES)]
            d = v - _MAIN_END
            is_x = (d >= 0) & ((d >> 7) == wid)
            own = ((v >= r0) & (v < r0 + _RANGE)) | is_x
            rel = jnp.where(is_x, _XREL + (d & 127), v - r0)
            pk = (rel << 14) | (c * _ICHUNK + g * _LANES + lane)
            plsc.store_compressed(own_v.at[pl.ds(cnt, _LANES)], pk, mask=own)
            pc = plsc.all_reduce_population_count(own)
            return cnt + lax.squeeze(lax.slice(pc, (0,), (1,)), (0,))

        return lax.fori_loop(0, _ICHUNK // _LANES, sel_body, cnt)

    cnt = lax.fori_loop(0, _BATCH // _ICHUNK, sel_chunk, jnp.int32(0))
    own_v[pl.ds(cnt, _LANES)] = lane * 0 + _SENTINEL
    ngroups = (cnt + _LANES - 1) >> 4

    def ring_slab(cb):
        slot = cb & (_EB - 1)
        return (
            ring_v.at[slot],
            out_hbm.at[jring_v.at[pl.ds(slot * _LANES, _LANES)]],
        )

    def process_wave(b, wb_rel, wn, carry):
        lo = wb_rel << 14
        hi = (wb_rel + wn) << 14

        def grp_body(e, c):
            egc, waits = c
            pkv = own_v[pl.ds(e * _LANES, _LANES)]
            inw = (pkv >= lo) & (pkv < hi)
            npc_v = plsc.all_reduce_population_count(inw)
            npc = lax.squeeze(lax.slice(npc_v, (0,), (1,)), (0,))
            fill = egc & (_LANES - 1)
            cb_old = egc >> 4
            need_wait = (fill + npc > _LANES) & (cb_old + 1 >= _EB)

            @pl.when(need_wait)
            def _wait_ring():
                src, dst = ring_slab(cb_old + 1)
                pltpu.make_async_copy(src, dst, sem_out).wait()

            @pl.when(npc > 0)
            def _extract():
                rank = plsc.cumsum(inw.astype(jnp.int32)) - 1
                pos = egc + rank
                slotv = (pos >> 4) & (_EB - 1)
                rowv = pos & (_LANES - 1)
                jv = pkv & (_BATCH - 1)
                local = (pkv >> 14) - wb_rel
                tcv = jnp.minimum((local >> 7) & 15, _WCOLS - 1)
                lv = local & 127
                bvec = lane * 0 + b
                for c in range(_DIM):
                    cvec = lane * 0 + c
                    vals = plsc.load_gather(
                        wave_v, [bvec, tcv, cvec, lv], mask=inw
                    )
                    plsc.store_scatter(
                        ring_v, [slotv, rowv, cvec], vals, mask=inw
                    )
                plsc.store_scatter(
                    jring_v, [pos & (_EB * _LANES - 1)], jv, mask=inw
                )

            @pl.when(fill + npc >= _LANES)
            def _fire_block():
                src, dst = ring_slab(cb_old)
                pltpu.async_copy(src, dst, sem_out)

            new_waits = jnp.where(need_wait, waits + 1, waits)
            return (egc + npc, new_waits)

        return lax.fori_loop(0, ngroups, grp_body, carry)

    def wave_body(k, carry):
        b = k & 1
        wait_wave(r0 + k * _WAVE, b, _WCOLS)
        carry = process_wave(b, k * _WAVE, _WAVE, carry)

        @pl.when(k < _NFULL - 2)
        def _prefetch():
            fire_wave(r0 + (k + 2) * _WAVE, b, _WCOLS)

        return carry

    carry = lax.fori_loop(0, _NFULL, wave_body, (jnp.int32(0), jnp.int32(0)))

    wb_tail = _NFULL * _WAVE
    fire_wave(r0 + wb_tail, 0, 4)
    wait_wave(r0 + wb_tail, 0, 4)
    carry = process_wave(0, wb_tail, 4 * 128, carry)

    fire_wave(xbase, 1, 1)
    wait_wave(xbase, 1, 1)
    carry = process_wave(1, _XREL, 128, carry)
    egc, waits = carry

    left = egc & (_LANES - 1)

    @pl.when(left > 0)
    def _flush():
        cb = egc >> 4
        slot = cb & (_EB - 1)
        jv_tail = jring_v[pl.ds(slot * _LANES, _LANES)]
        dummy = _BATCH + wid * _LANES + lane
        jring_v[pl.ds(slot * _LANES, _LANES)] = jnp.where(
            lane < left, jv_tail, dummy
        )
        src, dst = ring_slab(cb)
        pltpu.async_copy(src, dst, sem_out)

    total_fired = (egc >> 4) + jnp.where(left > 0, 1, 0).astype(jnp.int32)
    outstanding = total_fired - waits

    def drain_body(d, acc):
        src, dst = ring_slab(jnp.int32(0))
        pltpu.make_async_copy(src, dst, sem_out).wait()
        return acc

    lax.fori_loop(0, outstanding, drain_body, jnp.int32(0))


def kernel(shape_idx, emb_table):
    out_p = _gather_kernel(shape_idx.astype(jnp.int32), emb_table.T)
    return out_p[:_BATCH, :_DIM]
